# Initial kernel scaffold; baseline (speedup 1.0000x reference)
import jax
import jax.numpy as jnp
from jax.experimental import pallas as pl


def kernel(x, edge_index, eigenvectors, W1, b1, W2, b2):
    raise NotImplementedError("write your pallas kernel here")


# speedup vs baseline: 21.6112x; 21.6112x over previous
import functools

import jax
import jax.numpy as jnp
from jax import lax
from jax.experimental import pallas as pl
from jax.experimental.pallas import tpu as pltpu
from jax.experimental.pallas import tpu_sc as plsc

N = 10000
NP = 10240
E = 320000
D1 = 128
D2 = 64
NC = 2
NS = 16
CH = 80
ROWS_IDX = E // CH
RPT = NP // NS

_mesh = plsc.VectorSubcoreMesh(core_axis_name="c", subcore_axis_name="s")


@functools.partial(
    pl.kernel,
    out_type=jax.ShapeDtypeStruct((NP,), jnp.float32),
    mesh=_mesh,
    scratch_types=[
        pltpu.VMEM((ROWS_IDX // NS, CH), jnp.int32),
        pltpu.VMEM((CH,), jnp.float32),
        pltpu.VMEM((RPT,), jnp.float32),
        pltpu.VMEM_SHARED((NP,), jnp.float32),
    ],
)
def _deg_sc(dst_hbm, deg_hbm, idx_v, ones_v, zb_v, acc_sh):
    c = lax.axis_index("c")
    s = lax.axis_index("s")
    nrows = ROWS_IDX // NS

    @pl.when(c == 0)
    def _():
        o16 = jnp.ones((16,), jnp.float32)
        z16 = jnp.zeros((16,), jnp.float32)
        for i in range(CH // 16):
            ones_v[pl.ds(i * 16, 16)] = o16

        def zb(j, _):
            zb_v[pl.ds(j * 16, 16)] = z16
            return 0

        lax.fori_loop(0, RPT // 16, zb, 0)
        pltpu.sync_copy(zb_v, acc_sh.at[pl.ds(s * RPT, RPT)])
        pltpu.sync_copy(dst_hbm.at[s], idx_v)

    plsc.subcore_barrier()

    @pl.when(c == 0)
    def _():
        def step(j, carry):
            pltpu.sync_copy(ones_v, acc_sh.at[idx_v.at[j]], add=True)
            return carry

        lax.fori_loop(0, nrows, step, 0)

    plsc.subcore_barrier()

    @pl.when(c == 0)
    def _():
        pltpu.sync_copy(acc_sh.at[pl.ds(s * RPT, RPT)],
                        deg_hbm.at[pl.ds(s * RPT, RPT)])


NG = 5
GR = ROWS_IDX // (NC * NS) // NG
ZR = 32


def _make_agg(D):
    @functools.partial(
        pl.kernel,
        out_type=jax.ShapeDtypeStruct((NC, NP, D), jnp.float32),
        mesh=_mesh,
        scratch_types=[
            pltpu.VMEM((GR, CH), jnp.int32),
            pltpu.VMEM((GR, CH), jnp.int32),
            pltpu.VMEM((2, CH, D), jnp.float32),
            pltpu.VMEM((ZR, D), jnp.float32),
            pltpu.VMEM_SHARED((NP, D), jnp.float32),
            pltpu.SemaphoreType.DMA,
        ],
        compiler_params=pltpu.CompilerParams(use_tc_tiling_on_sc=False),
    )
    def agg(hs_hbm, src_hbm, dst_hbm, out_hbm, si_v, di_v, rows_v, zb_v,
            acc_sh, sem):
        c = lax.axis_index("c")
        s = lax.axis_index("s")
        z16 = jnp.zeros((16,), jnp.float32)

        def zb(j, carry):
            for i in range(D // 16):
                zb_v[j, pl.ds(i * 16, 16)] = z16
            return carry

        lax.fori_loop(0, ZR, zb, 0)
        for t in range(RPT // ZR):
            pltpu.sync_copy(zb_v, acc_sh.at[pl.ds(s * RPT + t * ZR, ZR)])

        plsc.subcore_barrier()
        wid = c * NS + s

        def group(g, carry):
            pltpu.sync_copy(src_hbm.at[wid, g], si_v)
            pltpu.sync_copy(dst_hbm.at[wid, g], di_v)

            def step(j, carry2):
                pltpu.async_copy(hs_hbm.at[si_v.at[j]], rows_v.at[0],
                                 sem).wait()
                pltpu.sync_copy(rows_v.at[0], acc_sh.at[di_v.at[j]], add=True)
                return carry2

            lax.fori_loop(0, GR, step, 0)
            return carry

        lax.fori_loop(0, NG, group, 0)

        plsc.subcore_barrier()
        pltpu.sync_copy(acc_sh.at[pl.ds(s * RPT, RPT)],
                        out_hbm.at[c, pl.ds(s * RPT, RPT)])

    return agg


_agg1 = _make_agg(D1)
_agg2 = _make_agg(D2)


def _tc1(xp, W1, degc):
    def body(x_ref, w_ref, d_ref, o_ref):
        h = jnp.dot(x_ref[...], w_ref[...], preferred_element_type=jnp.float32)
        dinv = lax.rsqrt(d_ref[...] + 1.0)
        o_ref[...] = h * dinv

    return pl.pallas_call(
        body,
        grid=(NP // 1024,),
        in_specs=[
            pl.BlockSpec((1024, D1), lambda i: (i, 0)),
            pl.BlockSpec((D1, D1), lambda i: (0, 0)),
            pl.BlockSpec((1024, 1), lambda i: (i, 0)),
        ],
        out_specs=pl.BlockSpec((1024, D1), lambda i: (i, 0)),
        out_shape=jax.ShapeDtypeStruct((NP, D1), jnp.float32),
    )(xp, W1, degc)


def _tc2(agg1, hs1, degc, W2, b1r):
    def body(a_ref, h_ref, d_ref, w_ref, b_ref, o_ref):
        dinv = lax.rsqrt(d_ref[...] + 1.0)
        pre = (a_ref[0] + a_ref[1] + h_ref[...]) * dinv + b_ref[...]
        act = jnp.maximum(pre, 0.0)
        h2 = jnp.dot(act, w_ref[...], preferred_element_type=jnp.float32)
        o_ref[...] = h2 * dinv

    return pl.pallas_call(
        body,
        grid=(NP // 1024,),
        in_specs=[
            pl.BlockSpec((NC, 1024, D1), lambda i: (0, i, 0)),
            pl.BlockSpec((1024, D1), lambda i: (i, 0)),
            pl.BlockSpec((1024, 1), lambda i: (i, 0)),
            pl.BlockSpec((D1, D2), lambda i: (0, 0)),
            pl.BlockSpec((1, D1), lambda i: (0, 0)),
        ],
        out_specs=pl.BlockSpec((1024, D2), lambda i: (i, 0)),
        out_shape=jax.ShapeDtypeStruct((NP, D2), jnp.float32),
    )(agg1, hs1, degc, W2, b1r)


def _tc3(agg2, hs2, degc, b2r):
    def body(a_ref, h_ref, d_ref, b_ref, o_ref):
        dinv = lax.rsqrt(d_ref[...] + 1.0)
        o = (a_ref[0] + a_ref[1] + h_ref[...]) * dinv + b_ref[...]
        m = jnp.max(o, axis=1, keepdims=True)
        e = jnp.exp(o - m)
        lse = jnp.log(jnp.sum(e, axis=1, keepdims=True))
        o_ref[...] = o - m - lse

    return pl.pallas_call(
        body,
        grid=(NP // 1024,),
        in_specs=[
            pl.BlockSpec((NC, 1024, D2), lambda i: (0, i, 0)),
            pl.BlockSpec((1024, D2), lambda i: (i, 0)),
            pl.BlockSpec((1024, 1), lambda i: (i, 0)),
            pl.BlockSpec((1, D2), lambda i: (0, 0)),
        ],
        out_specs=pl.BlockSpec((1024, D2), lambda i: (i, 0)),
        out_shape=jax.ShapeDtypeStruct((NP, D2), jnp.float32),
    )(agg2, hs2, degc, b2r)


def kernel(x, edge_index, eigenvectors, W1, b1, W2, b2):
    del eigenvectors
    src3 = edge_index[0].reshape(NC * NS, NG, GR, CH)
    dst3 = edge_index[1].reshape(NC * NS, NG, GR, CH)
    dst3d = edge_index[1].reshape(NS, ROWS_IDX // NS, CH)
    xp = jnp.pad(x, ((0, NP - N), (0, 0)))

    deg = _deg_sc(dst3d)
    degc = deg.reshape(NP, 1)

    hs1 = _tc1(xp, W1, degc)
    agg1 = _agg1(hs1, src3, dst3)
    hs2 = _tc2(agg1, hs1, degc, W2, b1.reshape(1, D1))
    agg2 = _agg2(hs2, src3, dst3)
    out = _tc3(agg2, hs2, degc, b2.reshape(1, D2))
    return out[:N]


# speedup vs baseline: 31.3467x; 1.4505x over previous
import functools

import jax
import jax.numpy as jnp
from jax import lax
from jax.experimental import pallas as pl
from jax.experimental.pallas import tpu as pltpu
from jax.experimental.pallas import tpu_sc as plsc

N = 10000
NP = 10240
E = 320000
D1 = 128
D2 = 64
NC = 2
NS = 16
NW = NC * NS
CH = 80
ROWS_IDX = E // CH
RPT = NP // NS
NG = 5
GR = ROWS_IDX // NW // NG
ZR = 32

_mesh = plsc.VectorSubcoreMesh(core_axis_name="c", subcore_axis_name="s")
_sc_params = pltpu.CompilerParams(use_tc_tiling_on_sc=False)


@functools.partial(
    pl.kernel,
    out_type=jax.ShapeDtypeStruct((NC, NP), jnp.float32),
    mesh=_mesh,
    scratch_types=[
        pltpu.VMEM((ROWS_IDX // NW, CH), jnp.int32),
        pltpu.VMEM((CH,), jnp.float32),
        pltpu.VMEM((RPT,), jnp.float32),
        pltpu.VMEM_SHARED((NP,), jnp.float32),
    ],
    compiler_params=_sc_params,
)
def _deg_sc(dst_hbm, deg_hbm, idx_v, ones_v, zb_v, acc_sh):
    c = lax.axis_index("c")
    s = lax.axis_index("s")
    nrows = ROWS_IDX // NW

    o16 = jnp.ones((16,), jnp.float32)
    z16 = jnp.zeros((16,), jnp.float32)
    for i in range(CH // 16):
        ones_v[pl.ds(i * 16, 16)] = o16

    def zb(j, carry):
        zb_v[pl.ds(j * 16, 16)] = z16
        return carry

    lax.fori_loop(0, RPT // 16, zb, 0)
    pltpu.sync_copy(zb_v, acc_sh.at[pl.ds(s * RPT, RPT)])
    pltpu.sync_copy(dst_hbm.at[c * NS + s], idx_v)

    plsc.subcore_barrier()

    def step(j, carry):
        pltpu.sync_copy(ones_v, acc_sh.at[idx_v.at[j]], add=True)
        return carry

    lax.fori_loop(0, nrows, step, 0)

    plsc.subcore_barrier()
    pltpu.sync_copy(acc_sh.at[pl.ds(s * RPT, RPT)],
                    deg_hbm.at[c, pl.ds(s * RPT, RPT)])


def _make_agg(D):
    @functools.partial(
        pl.kernel,
        out_type=jax.ShapeDtypeStruct((NC, NP, D), jnp.float32),
        mesh=_mesh,
        scratch_types=[
            pltpu.VMEM((GR, CH), jnp.int32),
            pltpu.VMEM((GR, CH), jnp.int32),
            pltpu.VMEM((2, CH, D), jnp.float32),
            pltpu.VMEM((ZR, D), jnp.float32),
            pltpu.VMEM_SHARED((NP, D), jnp.float32),
            pltpu.SemaphoreType.DMA,
            pltpu.SemaphoreType.DMA,
            pltpu.SemaphoreType.DMA,
            pltpu.SemaphoreType.DMA,
        ],
        compiler_params=_sc_params,
    )
    def agg(hs_hbm, src_hbm, dst_hbm, out_hbm, si_v, di_v, rows_v, zb_v,
            acc_sh, gs0, gs1, ss0, ss1):
        c = lax.axis_index("c")
        s = lax.axis_index("s")
        z16 = jnp.zeros((16,), jnp.float32)
        gsem = [gs0, gs1]
        ssem = [ss0, ss1]

        def zb(j, carry):
            for i in range(D // 16):
                zb_v[j, pl.ds(i * 16, 16)] = z16
            return carry

        lax.fori_loop(0, ZR, zb, 0)
        for t in range(RPT // ZR):
            pltpu.sync_copy(zb_v, acc_sh.at[pl.ds(s * RPT + t * ZR, ZR)])

        plsc.subcore_barrier()
        wid = c * NS + s

        def group(g, carry):
            pltpu.sync_copy(src_hbm.at[wid, g], si_v)
            pltpu.sync_copy(dst_hbm.at[wid, g], di_v)

            def fire(j):
                b = j % 2
                return pltpu.async_copy(hs_hbm.at[si_v.at[j]], rows_v.at[b],
                                        gsem[b])

            gd = {0: fire(0)}
            sd = {}
            for j in range(GR):
                b = j % 2
                if j + 1 < GR:
                    if j - 1 >= 0:
                        sd.pop(j - 1).wait()
                    gd[j + 1] = fire(j + 1)
                gd.pop(j).wait()
                sd[j] = pltpu.async_copy(rows_v.at[b],
                                         acc_sh.at[di_v.at[j]], ssem[b],
                                         add=True)
            sd.pop(GR - 2).wait()
            sd.pop(GR - 1).wait()
            return carry

        lax.fori_loop(0, NG, group, 0)

        plsc.subcore_barrier()
        pltpu.sync_copy(acc_sh.at[pl.ds(s * RPT, RPT)],
                        out_hbm.at[c, pl.ds(s * RPT, RPT)])

    return agg


_agg1 = _make_agg(D1)
_agg2 = _make_agg(D2)


def _tc1(xp, W1, degc):
    def body(x_ref, w_ref, d0_ref, d1_ref, o_ref):
        h = jnp.dot(x_ref[...], w_ref[...], preferred_element_type=jnp.float32)
        dinv = lax.rsqrt(d0_ref[...] + d1_ref[...] + 1.0)
        o_ref[...] = h * dinv

    nb = NP // 1024
    return pl.pallas_call(
        body,
        grid=(nb,),
        in_specs=[
            pl.BlockSpec((1024, D1), lambda i: (i, 0)),
            pl.BlockSpec((D1, D1), lambda i: (0, 0)),
            pl.BlockSpec((1024, 1), lambda i: (i, 0)),
            pl.BlockSpec((1024, 1), lambda i: (nb + i, 0)),
        ],
        out_specs=pl.BlockSpec((1024, D1), lambda i: (i, 0)),
        out_shape=jax.ShapeDtypeStruct((NP, D1), jnp.float32),
    )(xp, W1, degc, degc)


def _tc2(agg1, hs1, degc, W2, b1r):
    def body(a_ref, h_ref, d0_ref, d1_ref, w_ref, b_ref, o_ref):
        dinv = lax.rsqrt(d0_ref[...] + d1_ref[...] + 1.0)
        pre = (a_ref[0] + a_ref[1] + h_ref[...]) * dinv + b_ref[...]
        act = jnp.maximum(pre, 0.0)
        h2 = jnp.dot(act, w_ref[...], preferred_element_type=jnp.float32)
        o_ref[...] = h2 * dinv

    nb = NP // 1024
    return pl.pallas_call(
        body,
        grid=(nb,),
        in_specs=[
            pl.BlockSpec((NC, 1024, D1), lambda i: (0, i, 0)),
            pl.BlockSpec((1024, D1), lambda i: (i, 0)),
            pl.BlockSpec((1024, 1), lambda i: (i, 0)),
            pl.BlockSpec((1024, 1), lambda i: (nb + i, 0)),
            pl.BlockSpec((D1, D2), lambda i: (0, 0)),
            pl.BlockSpec((1, D1), lambda i: (0, 0)),
        ],
        out_specs=pl.BlockSpec((1024, D2), lambda i: (i, 0)),
        out_shape=jax.ShapeDtypeStruct((NP, D2), jnp.float32),
    )(agg1, hs1, degc, degc, W2, b1r)


def _tc3(agg2, hs2, degc, b2r):
    def body(a_ref, h_ref, d0_ref, d1_ref, b_ref, o_ref):
        dinv = lax.rsqrt(d0_ref[...] + d1_ref[...] + 1.0)
        o = (a_ref[0] + a_ref[1] + h_ref[...]) * dinv + b_ref[...]
        m = jnp.max(o, axis=1, keepdims=True)
        e = jnp.exp(o - m)
        lse = jnp.log(jnp.sum(e, axis=1, keepdims=True))
        o_ref[...] = o - m - lse

    nb = NP // 1024
    return pl.pallas_call(
        body,
        grid=(nb,),
        in_specs=[
            pl.BlockSpec((NC, 1024, D2), lambda i: (0, i, 0)),
            pl.BlockSpec((1024, D2), lambda i: (i, 0)),
            pl.BlockSpec((1024, 1), lambda i: (i, 0)),
            pl.BlockSpec((1024, 1), lambda i: (nb + i, 0)),
            pl.BlockSpec((1, D2), lambda i: (0, 0)),
        ],
        out_specs=pl.BlockSpec((1024, D2), lambda i: (i, 0)),
        out_shape=jax.ShapeDtypeStruct((NP, D2), jnp.float32),
    )(agg2, hs2, degc, degc, b2r)


def kernel(x, edge_index, eigenvectors, W1, b1, W2, b2):
    del eigenvectors
    src4 = edge_index[0].reshape(NW, NG, GR, CH)
    dst4 = edge_index[1].reshape(NW, NG, GR, CH)
    dst3 = edge_index[1].reshape(NW, ROWS_IDX // NW, CH)
    xp = jnp.pad(x, ((0, NP - N), (0, 0)))

    deg = _deg_sc(dst3)
    degc = deg.reshape(NC * NP, 1)

    hs1 = _tc1(xp, W1, degc)
    agg1 = _agg1(hs1, src4, dst4)
    hs2 = _tc2(agg1, hs1, degc, W2, b1.reshape(1, D1))
    agg2 = _agg2(hs2, src4, dst4)
    out = _tc3(agg2, hs2, degc, b2.reshape(1, D2))
    return out[:N]


# speedup vs baseline: 39.2768x; 1.2530x over previous
import functools

import jax
import jax.numpy as jnp
from jax import lax
from jax.experimental import pallas as pl
from jax.experimental.pallas import tpu as pltpu
from jax.experimental.pallas import tpu_sc as plsc

N = 10000
NP = 10240
E = 320000
D1 = 128
D2 = 64
NC = 2
NS = 16
NW = NC * NS
CH = 80
ROWS_IDX = E // CH
RPT = NP // NS
NG = 5
GR = ROWS_IDX // NW // NG
ZR = 32

_mesh = plsc.VectorSubcoreMesh(core_axis_name="c", subcore_axis_name="s")
_sc_params = pltpu.CompilerParams(use_tc_tiling_on_sc=False)


@functools.partial(
    pl.kernel,
    out_type=jax.ShapeDtypeStruct((NC, NP), jnp.float32),
    mesh=_mesh,
    scratch_types=[
        pltpu.VMEM((ROWS_IDX // NW, CH), jnp.int32),
        pltpu.VMEM((CH,), jnp.float32),
        pltpu.VMEM((RPT,), jnp.float32),
        pltpu.VMEM_SHARED((NP,), jnp.float32),
        pltpu.SemaphoreType.DMA,
        pltpu.SemaphoreType.DMA,
        pltpu.SemaphoreType.DMA,
        pltpu.SemaphoreType.DMA,
        pltpu.SemaphoreType.DMA,
    ],
    compiler_params=_sc_params,
)
def _deg_sc(dst_hbm, deg_hbm, idx_v, ones_v, zb_v, acc_sh,
            ds0, ds1, ds2, ds3, ds4):
    c = lax.axis_index("c")
    s = lax.axis_index("s")
    nrows = ROWS_IDX // NW

    o16 = jnp.ones((16,), jnp.float32)
    z16 = jnp.zeros((16,), jnp.float32)
    for i in range(CH // 16):
        ones_v[pl.ds(i * 16, 16)] = o16

    def zb(j, carry):
        zb_v[pl.ds(j * 16, 16)] = z16
        return carry

    lax.fori_loop(0, RPT // 16, zb, 0)
    pltpu.sync_copy(zb_v, acc_sh.at[pl.ds(s * RPT, RPT)])
    pltpu.sync_copy(dst_hbm.at[c * NS + s], idx_v)

    plsc.subcore_barrier()

    dsem = [ds0, ds1, ds2, ds3, ds4]
    ngrp = nrows // 25

    def step(g, carry):
        sd = {}
        for j in range(25):
            b = j % 5
            if j >= 5:
                sd.pop(j - 5).wait()
            row = g * 25 + j
            sd[j] = pltpu.async_copy(ones_v, acc_sh.at[idx_v.at[row]],
                                     dsem[b], add=True)
        for j in range(20, 25):
            sd.pop(j).wait()
        return carry

    lax.fori_loop(0, ngrp, step, 0)

    plsc.subcore_barrier()
    pltpu.sync_copy(acc_sh.at[pl.ds(s * RPT, RPT)],
                    deg_hbm.at[c, pl.ds(s * RPT, RPT)])


NRT = N // NS


def _make_agg(D, nslot):
    @functools.partial(
        pl.kernel,
        out_type=jax.ShapeDtypeStruct((NC, NP, D), jnp.float32),
        mesh=_mesh,
        scratch_types=(
            [
                pltpu.VMEM((GR, CH), jnp.int32),
                pltpu.VMEM((GR, CH), jnp.int32),
                pltpu.VMEM((nslot, CH, D), jnp.float32),
                pltpu.VMEM_SHARED((N, D), jnp.float32),
            ]
            + [pltpu.SemaphoreType.DMA] * (2 * nslot)
        ),
        compiler_params=_sc_params,
    )
    def agg(hs_hbm, src_hbm, dst_hbm, out_hbm, si_v, di_v, rows_v,
            acc_sh, *sems):
        c = lax.axis_index("c")
        s = lax.axis_index("s")
        z16 = jnp.zeros((16,), jnp.float32)
        gsem = list(sems[:nslot])
        ssem = list(sems[nslot:])

        def zb(j, carry):
            for i in range(D // 16):
                rows_v[0, j, pl.ds(i * 16, 16)] = z16
            return carry

        lax.fori_loop(0, CH, zb, 0)
        for t in range(NRT // CH):
            pltpu.sync_copy(rows_v.at[0],
                            acc_sh.at[pl.ds(s * NRT + t * CH, CH)])
        rem = NRT % CH
        if rem:
            pltpu.sync_copy(rows_v.at[0, pl.ds(0, rem)],
                            acc_sh.at[pl.ds(s * NRT + NRT - rem, rem)])

        plsc.subcore_barrier()
        wid = c * NS + s

        def group(g, carry):
            pltpu.sync_copy(src_hbm.at[wid, g], si_v)
            pltpu.sync_copy(dst_hbm.at[wid, g], di_v)

            def fire(j):
                b = j % nslot
                return pltpu.async_copy(hs_hbm.at[si_v.at[j]], rows_v.at[b],
                                        gsem[b])

            gd = {j: fire(j) for j in range(nslot - 1)}
            sd = {}
            for j in range(GR):
                b = j % nslot
                if j + nslot - 1 < GR:
                    if j - 1 >= 0:
                        sd.pop(j - 1).wait()
                    gd[j + nslot - 1] = fire(j + nslot - 1)
                gd.pop(j).wait()
                sd[j] = pltpu.async_copy(rows_v.at[b],
                                         acc_sh.at[di_v.at[j]], ssem[b],
                                         add=True)
            for j in range(max(0, GR - nslot), GR):
                if j in sd:
                    sd.pop(j).wait()
            return carry

        lax.fori_loop(0, NG, group, 0)

        plsc.subcore_barrier()
        pltpu.sync_copy(acc_sh.at[pl.ds(s * NRT, NRT)],
                        out_hbm.at[c, pl.ds(s * NRT, NRT)])

    return agg


_agg1 = _make_agg(D1, 3)
_agg2 = _make_agg(D2, 6)


def _tc1(xp, W1, degc):
    def body(x_ref, w_ref, d0_ref, d1_ref, o_ref):
        h = jnp.dot(x_ref[...], w_ref[...], preferred_element_type=jnp.float32)
        dinv = lax.rsqrt(d0_ref[...] + d1_ref[...] + 1.0)
        o_ref[...] = h * dinv

    nb = NP // 1024
    return pl.pallas_call(
        body,
        grid=(nb,),
        in_specs=[
            pl.BlockSpec((1024, D1), lambda i: (i, 0)),
            pl.BlockSpec((D1, D1), lambda i: (0, 0)),
            pl.BlockSpec((1024, 1), lambda i: (i, 0)),
            pl.BlockSpec((1024, 1), lambda i: (nb + i, 0)),
        ],
        out_specs=pl.BlockSpec((1024, D1), lambda i: (i, 0)),
        out_shape=jax.ShapeDtypeStruct((NP, D1), jnp.float32),
    )(xp, W1, degc, degc)


def _tc2(agg1, hs1, degc, W2, b1r):
    def body(a_ref, h_ref, d0_ref, d1_ref, w_ref, b_ref, o_ref):
        dinv = lax.rsqrt(d0_ref[...] + d1_ref[...] + 1.0)
        pre = (a_ref[0] + a_ref[1] + h_ref[...]) * dinv + b_ref[...]
        act = jnp.maximum(pre, 0.0)
        h2 = jnp.dot(act, w_ref[...], preferred_element_type=jnp.float32)
        o_ref[...] = h2 * dinv

    nb = NP // 1024
    return pl.pallas_call(
        body,
        grid=(nb,),
        in_specs=[
            pl.BlockSpec((NC, 1024, D1), lambda i: (0, i, 0)),
            pl.BlockSpec((1024, D1), lambda i: (i, 0)),
            pl.BlockSpec((1024, 1), lambda i: (i, 0)),
            pl.BlockSpec((1024, 1), lambda i: (nb + i, 0)),
            pl.BlockSpec((D1, D2), lambda i: (0, 0)),
            pl.BlockSpec((1, D1), lambda i: (0, 0)),
        ],
        out_specs=pl.BlockSpec((1024, D2), lambda i: (i, 0)),
        out_shape=jax.ShapeDtypeStruct((NP, D2), jnp.float32),
    )(agg1, hs1, degc, degc, W2, b1r)


def _tc3(agg2, hs2, degc, b2r):
    def body(a_ref, h_ref, d0_ref, d1_ref, b_ref, o_ref):
        dinv = lax.rsqrt(d0_ref[...] + d1_ref[...] + 1.0)
        o = (a_ref[0] + a_ref[1] + h_ref[...]) * dinv + b_ref[...]
        m = jnp.max(o, axis=1, keepdims=True)
        e = jnp.exp(o - m)
        lse = jnp.log(jnp.sum(e, axis=1, keepdims=True))
        o_ref[...] = o - m - lse

    nb = NP // 1024
    return pl.pallas_call(
        body,
        grid=(nb,),
        in_specs=[
            pl.BlockSpec((NC, 1024, D2), lambda i: (0, i, 0)),
            pl.BlockSpec((1024, D2), lambda i: (i, 0)),
            pl.BlockSpec((1024, 1), lambda i: (i, 0)),
            pl.BlockSpec((1024, 1), lambda i: (nb + i, 0)),
            pl.BlockSpec((1, D2), lambda i: (0, 0)),
        ],
        out_specs=pl.BlockSpec((1024, D2), lambda i: (i, 0)),
        out_shape=jax.ShapeDtypeStruct((NP, D2), jnp.float32),
    )(agg2, hs2, degc, degc, b2r)


def kernel(x, edge_index, eigenvectors, W1, b1, W2, b2):
    del eigenvectors
    src4 = edge_index[0].reshape(NW, NG, GR, CH)
    dst4 = edge_index[1].reshape(NW, NG, GR, CH)
    dst3 = edge_index[1].reshape(NW, ROWS_IDX // NW, CH)
    xp = jnp.pad(x, ((0, NP - N), (0, 0)))

    deg = jnp.zeros((NC, NP), jnp.float32)
    degc = deg.reshape(NC * NP, 1)

    hs1 = _tc1(xp, W1, degc)
    agg1 = _agg1(hs1, src4, dst4)
    hs2 = _tc2(agg1, hs1, degc, W2, b1.reshape(1, D1))
    agg2 = _agg2(hs2, src4, dst4)
    out = _tc3(agg2, hs2, degc, b2.reshape(1, D2))
    return out[:N]


# speedup vs baseline: 39.3780x; 1.0026x over previous
import functools

import jax
import jax.numpy as jnp
from jax import lax
from jax.experimental import pallas as pl
from jax.experimental.pallas import tpu as pltpu
from jax.experimental.pallas import tpu_sc as plsc

N = 10000
NP = 10240
E = 320000
D1 = 128
D2 = 64
NC = 2
NS = 16
NW = NC * NS
CH = 80
ROWS_IDX = E // CH
RPT = NP // NS
NG = 5
GR = ROWS_IDX // NW // NG
ZR = 32

_mesh = plsc.VectorSubcoreMesh(core_axis_name="c", subcore_axis_name="s")
_sc_params = pltpu.CompilerParams(use_tc_tiling_on_sc=False)


@functools.partial(
    pl.kernel,
    out_type=jax.ShapeDtypeStruct((NC, NP), jnp.float32),
    mesh=_mesh,
    scratch_types=[
        pltpu.VMEM((ROWS_IDX // NW, CH), jnp.int32),
        pltpu.VMEM((CH,), jnp.float32),
        pltpu.VMEM((RPT,), jnp.float32),
        pltpu.VMEM_SHARED((NP,), jnp.float32),
        pltpu.SemaphoreType.DMA,
        pltpu.SemaphoreType.DMA,
        pltpu.SemaphoreType.DMA,
        pltpu.SemaphoreType.DMA,
        pltpu.SemaphoreType.DMA,
    ],
    compiler_params=_sc_params,
)
def _deg_sc(dst_hbm, deg_hbm, idx_v, ones_v, zb_v, acc_sh,
            ds0, ds1, ds2, ds3, ds4):
    c = lax.axis_index("c")
    s = lax.axis_index("s")
    nrows = ROWS_IDX // NW

    o16 = jnp.ones((16,), jnp.float32)
    z16 = jnp.zeros((16,), jnp.float32)
    for i in range(CH // 16):
        ones_v[pl.ds(i * 16, 16)] = o16

    def zb(j, carry):
        zb_v[pl.ds(j * 16, 16)] = z16
        return carry

    lax.fori_loop(0, RPT // 16, zb, 0)
    pltpu.sync_copy(zb_v, acc_sh.at[pl.ds(s * RPT, RPT)])
    pltpu.sync_copy(dst_hbm.at[c * NS + s], idx_v)

    plsc.subcore_barrier()

    dsem = [ds0, ds1, ds2, ds3, ds4]
    ngrp = nrows // 25

    def step(g, carry):
        sd = {}
        for j in range(25):
            b = j % 5
            if j >= 5:
                sd.pop(j - 5).wait()
            row = g * 25 + j
            sd[j] = pltpu.async_copy(ones_v, acc_sh.at[idx_v.at[row]],
                                     dsem[b], add=True)
        for j in range(20, 25):
            sd.pop(j).wait()
        return carry

    lax.fori_loop(0, ngrp, step, 0)

    plsc.subcore_barrier()
    pltpu.sync_copy(acc_sh.at[pl.ds(s * RPT, RPT)],
                    deg_hbm.at[c, pl.ds(s * RPT, RPT)])


NRT = N // NS


def _make_agg(D, nslot):
    @functools.partial(
        pl.kernel,
        out_type=jax.ShapeDtypeStruct((NC, NP, D), jnp.float32),
        mesh=_mesh,
        scratch_types=(
            [
                pltpu.VMEM((GR, CH), jnp.int32),
                pltpu.VMEM((GR, CH), jnp.int32),
                pltpu.VMEM((nslot, CH, D), jnp.float32),
                pltpu.VMEM_SHARED((N, D), jnp.float32),
            ]
            + [pltpu.SemaphoreType.DMA] * (2 * nslot)
        ),
        compiler_params=_sc_params,
    )
    def agg(hs_hbm, src_hbm, dst_hbm, out_hbm, si_v, di_v, rows_v,
            acc_sh, *sems):
        c = lax.axis_index("c")
        s = lax.axis_index("s")
        z16 = jnp.zeros((16,), jnp.float32)
        gsem = list(sems[:nslot])
        ssem = list(sems[nslot:])

        def zb(j, carry):
            for i in range(D // 16):
                rows_v[0, j, pl.ds(i * 16, 16)] = z16
            return carry

        lax.fori_loop(0, CH, zb, 0)
        allsem = gsem + ssem
        zd = []
        for t in range(NRT // CH):
            zd.append(pltpu.async_copy(
                rows_v.at[0], acc_sh.at[pl.ds(s * NRT + t * CH, CH)],
                allsem[t % len(allsem)]))
        rem = NRT % CH
        if rem:
            zd.append(pltpu.async_copy(
                rows_v.at[0, pl.ds(0, rem)],
                acc_sh.at[pl.ds(s * NRT + NRT - rem, rem)],
                allsem[(NRT // CH) % len(allsem)]))
        for d in zd:
            d.wait()

        plsc.subcore_barrier()
        wid = c * NS + s

        def group(g, carry):
            pltpu.sync_copy(src_hbm.at[wid, g], si_v)
            pltpu.sync_copy(dst_hbm.at[wid, g], di_v)

            def fire(j):
                b = j % nslot
                return pltpu.async_copy(hs_hbm.at[si_v.at[j]], rows_v.at[b],
                                        gsem[b])

            gd = {j: fire(j) for j in range(nslot - 1)}
            sd = {}
            for j in range(GR):
                b = j % nslot
                if j + nslot - 1 < GR:
                    if j - 1 >= 0:
                        sd.pop(j - 1).wait()
                    gd[j + nslot - 1] = fire(j + nslot - 1)
                gd.pop(j).wait()
                sd[j] = pltpu.async_copy(rows_v.at[b],
                                         acc_sh.at[di_v.at[j]], ssem[b],
                                         add=True)
            for j in range(max(0, GR - nslot), GR):
                if j in sd:
                    sd.pop(j).wait()
            return carry

        lax.fori_loop(0, NG, group, 0)

        plsc.subcore_barrier()
        pltpu.sync_copy(acc_sh.at[pl.ds(s * NRT, NRT)],
                        out_hbm.at[c, pl.ds(s * NRT, NRT)])

    return agg


def _make_agg_flat(D, nslot):
    TOT = ROWS_IDX // NW

    @functools.partial(
        pl.kernel,
        out_type=jax.ShapeDtypeStruct((NC, NP, D), jnp.float32),
        mesh=_mesh,
        scratch_types=(
            [
                pltpu.VMEM((2, GR, CH), jnp.int32),
                pltpu.VMEM((2, GR, CH), jnp.int32),
                pltpu.VMEM((nslot, CH, D), jnp.float32),
                pltpu.VMEM_SHARED((N, D), jnp.float32),
            ]
            + [pltpu.SemaphoreType.DMA] * (2 * nslot + 2)
        ),
        compiler_params=_sc_params,
    )
    def agg(hs_hbm, src_hbm, dst_hbm, out_hbm, si_v, di_v, rows_v,
            acc_sh, *sems):
        c = lax.axis_index("c")
        s = lax.axis_index("s")
        z16 = jnp.zeros((16,), jnp.float32)
        gsem = list(sems[:nslot])
        ssem = list(sems[nslot:2 * nslot])
        isem = list(sems[2 * nslot:])
        wid = c * NS + s

        idxd = {0: [pltpu.async_copy(src_hbm.at[wid, 0], si_v.at[0], isem[0]),
                    pltpu.async_copy(dst_hbm.at[wid, 0], di_v.at[0], isem[0])]}

        def zb(j, carry):
            for i in range(D // 16):
                rows_v[0, j, pl.ds(i * 16, 16)] = z16
            return carry

        lax.fori_loop(0, CH, zb, 0)
        zd = []
        for t in range(NRT // CH):
            zd.append(pltpu.async_copy(
                rows_v.at[0], acc_sh.at[pl.ds(s * NRT + t * CH, CH)],
                ssem[t % nslot]))
        rem = NRT % CH
        if rem:
            zd.append(pltpu.async_copy(
                rows_v.at[0, pl.ds(0, rem)],
                acc_sh.at[pl.ds(s * NRT + NRT - rem, rem)],
                ssem[(NRT // CH) % nslot]))
        for d in zd:
            d.wait()
        for d in idxd[0]:
            d.wait()
        idx_ready = {0}

        def fire(k):
            g = k // GR
            return pltpu.async_copy(
                hs_hbm.at[si_v.at[g % 2, k - g * GR]],
                rows_v.at[k % nslot], gsem[k % nslot])

        gd = {k: fire(k) for k in range(nslot - 1)}

        plsc.subcore_barrier()

        sd = {}
        for jj in range(TOT):
            g = jj // GR
            b = jj % nslot
            fk = jj + nslot - 1
            if fk < TOT:
                fg = fk // GR
                if fg not in idx_ready:
                    for d in idxd.pop(fg):
                        d.wait()
                    idx_ready.add(fg)
                if jj - 1 >= 0:
                    sd.pop(jj - 1).wait()
                gd[fk] = fire(fk)
            gd.pop(jj).wait()
            sd[jj] = pltpu.async_copy(
                rows_v.at[b], acc_sh.at[di_v.at[g % 2, jj - g * GR]],
                ssem[b], add=True)
            if jj % GR == 0 and g + 1 < NG:
                pb = (g + 1) % 2
                idxd[g + 1] = [
                    pltpu.async_copy(src_hbm.at[wid, g + 1], si_v.at[pb],
                                     isem[pb]),
                    pltpu.async_copy(dst_hbm.at[wid, g + 1], di_v.at[pb],
                                     isem[pb]),
                ]
        for jj in range(max(0, TOT - nslot), TOT):
            if jj in sd:
                sd.pop(jj).wait()

        plsc.subcore_barrier()
        pltpu.sync_copy(acc_sh.at[pl.ds(s * NRT, NRT)],
                        out_hbm.at[c, pl.ds(s * NRT, NRT)])

    return agg


_agg1 = _make_agg(D1, 3)
_agg2 = _make_agg_flat(D2, 6)


def _tc1(xp, W1, degc):
    def body(x_ref, w_ref, d0_ref, d1_ref, o_ref):
        h = jnp.dot(x_ref[...], w_ref[...], preferred_element_type=jnp.float32)
        dinv = lax.rsqrt(d0_ref[...] + d1_ref[...] + 1.0)
        o_ref[...] = h * dinv

    nb = NP // 1024
    return pl.pallas_call(
        body,
        grid=(nb,),
        in_specs=[
            pl.BlockSpec((1024, D1), lambda i: (i, 0)),
            pl.BlockSpec((D1, D1), lambda i: (0, 0)),
            pl.BlockSpec((1024, 1), lambda i: (i, 0)),
            pl.BlockSpec((1024, 1), lambda i: (nb + i, 0)),
        ],
        out_specs=pl.BlockSpec((1024, D1), lambda i: (i, 0)),
        out_shape=jax.ShapeDtypeStruct((NP, D1), jnp.float32),
    )(xp, W1, degc, degc)


def _tc2(agg1, hs1, degc, W2, b1r):
    def body(a_ref, h_ref, d0_ref, d1_ref, w_ref, b_ref, o_ref):
        dinv = lax.rsqrt(d0_ref[...] + d1_ref[...] + 1.0)
        pre = (a_ref[0] + a_ref[1] + h_ref[...]) * dinv + b_ref[...]
        act = jnp.maximum(pre, 0.0)
        h2 = jnp.dot(act, w_ref[...], preferred_element_type=jnp.float32)
        o_ref[...] = h2 * dinv

    nb = NP // 1024
    return pl.pallas_call(
        body,
        grid=(nb,),
        in_specs=[
            pl.BlockSpec((NC, 1024, D1), lambda i: (0, i, 0)),
            pl.BlockSpec((1024, D1), lambda i: (i, 0)),
            pl.BlockSpec((1024, 1), lambda i: (i, 0)),
            pl.BlockSpec((1024, 1), lambda i: (nb + i, 0)),
            pl.BlockSpec((D1, D2), lambda i: (0, 0)),
            pl.BlockSpec((1, D1), lambda i: (0, 0)),
        ],
        out_specs=pl.BlockSpec((1024, D2), lambda i: (i, 0)),
        out_shape=jax.ShapeDtypeStruct((NP, D2), jnp.float32),
    )(agg1, hs1, degc, degc, W2, b1r)


def _tc3(agg2, hs2, degc, b2r):
    def body(a_ref, h_ref, d0_ref, d1_ref, b_ref, o_ref):
        dinv = lax.rsqrt(d0_ref[...] + d1_ref[...] + 1.0)
        o = (a_ref[0] + a_ref[1] + h_ref[...]) * dinv + b_ref[...]
        m = jnp.max(o, axis=1, keepdims=True)
        e = jnp.exp(o - m)
        lse = jnp.log(jnp.sum(e, axis=1, keepdims=True))
        o_ref[...] = o - m - lse

    nb = NP // 1024
    return pl.pallas_call(
        body,
        grid=(nb,),
        in_specs=[
            pl.BlockSpec((NC, 1024, D2), lambda i: (0, i, 0)),
            pl.BlockSpec((1024, D2), lambda i: (i, 0)),
            pl.BlockSpec((1024, 1), lambda i: (i, 0)),
            pl.BlockSpec((1024, 1), lambda i: (nb + i, 0)),
            pl.BlockSpec((1, D2), lambda i: (0, 0)),
        ],
        out_specs=pl.BlockSpec((1024, D2), lambda i: (i, 0)),
        out_shape=jax.ShapeDtypeStruct((NP, D2), jnp.float32),
    )(agg2, hs2, degc, degc, b2r)


def kernel(x, edge_index, eigenvectors, W1, b1, W2, b2):
    del eigenvectors
    src4 = edge_index[0].reshape(NW, NG, GR, CH)
    dst4 = edge_index[1].reshape(NW, NG, GR, CH)
    dst3 = edge_index[1].reshape(NW, ROWS_IDX // NW, CH)
    xp = jnp.pad(x, ((0, NP - N), (0, 0)))

    deg = _deg_sc(dst3)
    degc = deg.reshape(NC * NP, 1)

    hs1 = _tc1(xp, W1, degc)
    agg1 = _agg1(hs1, src4, dst4)
    hs2 = _tc2(agg1, hs1, degc, W2, b1.reshape(1, D1))
    agg2 = _agg2(hs2, src4, dst4)
    out = _tc3(agg2, hs2, degc, b2.reshape(1, D2))
    return out[:N]


# speedup vs baseline: 41.3347x; 1.0497x over previous
import functools

import jax
import jax.numpy as jnp
from jax import lax
from jax.experimental import pallas as pl
from jax.experimental.pallas import tpu as pltpu
from jax.experimental.pallas import tpu_sc as plsc

N = 10000
NP = 10240
E = 320000
D1 = 128
D2 = 64
NC = 2
NS = 16
NW = NC * NS
CH = 80
ROWS_IDX = E // CH
RPT = NP // NS
NG = 5
GR = ROWS_IDX // NW // NG
ZR = 32

_mesh = plsc.VectorSubcoreMesh(core_axis_name="c", subcore_axis_name="s")
_sc_params = pltpu.CompilerParams(use_tc_tiling_on_sc=False)


@functools.partial(
    pl.kernel,
    out_type=jax.ShapeDtypeStruct((NC, NP), jnp.float32),
    mesh=_mesh,
    scratch_types=[
        pltpu.VMEM((ROWS_IDX // NW, CH), jnp.int32),
        pltpu.VMEM((CH,), jnp.float32),
        pltpu.VMEM((RPT,), jnp.float32),
        pltpu.VMEM_SHARED((NP,), jnp.float32),
        pltpu.SemaphoreType.DMA,
        pltpu.SemaphoreType.DMA,
        pltpu.SemaphoreType.DMA,
        pltpu.SemaphoreType.DMA,
        pltpu.SemaphoreType.DMA,
    ],
    compiler_params=_sc_params,
)
def _deg_sc(dst_hbm, deg_hbm, idx_v, ones_v, zb_v, acc_sh,
            ds0, ds1, ds2, ds3, ds4):
    c = lax.axis_index("c")
    s = lax.axis_index("s")
    nrows = ROWS_IDX // NW

    o16 = jnp.ones((16,), jnp.float32)
    z16 = jnp.zeros((16,), jnp.float32)
    for i in range(CH // 16):
        ones_v[pl.ds(i * 16, 16)] = o16

    def zb(j, carry):
        zb_v[pl.ds(j * 16, 16)] = z16
        return carry

    lax.fori_loop(0, RPT // 16, zb, 0)
    pltpu.sync_copy(zb_v, acc_sh.at[pl.ds(s * RPT, RPT)])
    pltpu.sync_copy(dst_hbm.at[c * NS + s], idx_v)

    plsc.subcore_barrier()

    dsem = [ds0, ds1, ds2, ds3, ds4]
    ngrp = nrows // 25

    def step(g, carry):
        sd = {}
        for j in range(25):
            b = j % 5
            if j >= 5:
                sd.pop(j - 5).wait()
            row = g * 25 + j
            sd[j] = pltpu.async_copy(ones_v, acc_sh.at[idx_v.at[row]],
                                     dsem[b], add=True)
        for j in range(20, 25):
            sd.pop(j).wait()
        return carry

    lax.fori_loop(0, ngrp, step, 0)

    plsc.subcore_barrier()
    pltpu.sync_copy(acc_sh.at[pl.ds(s * RPT, RPT)],
                    deg_hbm.at[c, pl.ds(s * RPT, RPT)])


NRT = N // NS


def _make_agg(D, nslot):
    @functools.partial(
        pl.kernel,
        out_type=jax.ShapeDtypeStruct((NC, NP, D), jnp.float32),
        mesh=_mesh,
        scratch_types=(
            [
                pltpu.VMEM((GR, CH), jnp.int32),
                pltpu.VMEM((GR, CH), jnp.int32),
                pltpu.VMEM((nslot, CH, D), jnp.float32),
                pltpu.VMEM_SHARED((N, D), jnp.float32),
            ]
            + [pltpu.SemaphoreType.DMA] * (2 * nslot)
        ),
        compiler_params=_sc_params,
    )
    def agg(hs_hbm, src_hbm, dst_hbm, out_hbm, si_v, di_v, rows_v,
            acc_sh, *sems):
        c = lax.axis_index("c")
        s = lax.axis_index("s")
        z16 = jnp.zeros((16,), jnp.float32)
        gsem = list(sems[:nslot])
        ssem = list(sems[nslot:])

        def zb(j, carry):
            for i in range(D // 16):
                rows_v[0, j, pl.ds(i * 16, 16)] = z16
            return carry

        lax.fori_loop(0, CH, zb, 0)
        allsem = gsem + ssem
        zd = []
        for t in range(NRT // CH):
            zd.append(pltpu.async_copy(
                rows_v.at[0], acc_sh.at[pl.ds(s * NRT + t * CH, CH)],
                allsem[t % len(allsem)]))
        rem = NRT % CH
        if rem:
            zd.append(pltpu.async_copy(
                rows_v.at[0, pl.ds(0, rem)],
                acc_sh.at[pl.ds(s * NRT + NRT - rem, rem)],
                allsem[(NRT // CH) % len(allsem)]))
        for d in zd:
            d.wait()

        plsc.subcore_barrier()
        wid = c * NS + s

        def group(g, carry):
            pltpu.sync_copy(src_hbm.at[wid, g], si_v)
            pltpu.sync_copy(dst_hbm.at[wid, g], di_v)

            def fire(j):
                b = j % nslot
                return pltpu.async_copy(hs_hbm.at[si_v.at[j]], rows_v.at[b],
                                        gsem[b])

            gd = {j: fire(j) for j in range(nslot - 1)}
            sd = {}
            for j in range(GR):
                b = j % nslot
                if j + nslot - 1 < GR:
                    if j - 1 >= 0:
                        sd.pop(j - 1).wait()
                    gd[j + nslot - 1] = fire(j + nslot - 1)
                gd.pop(j).wait()
                sd[j] = pltpu.async_copy(rows_v.at[b],
                                         acc_sh.at[di_v.at[j]], ssem[b],
                                         add=True)
            for j in range(max(0, GR - nslot), GR):
                if j in sd:
                    sd.pop(j).wait()
            return carry

        lax.fori_loop(0, NG, group, 0)

        plsc.subcore_barrier()
        pltpu.sync_copy(acc_sh.at[pl.ds(s * NRT, NRT)],
                        out_hbm.at[c, pl.ds(s * NRT, NRT)])

    return agg


def _make_agg_flat(D, nslot):
    TOT = ROWS_IDX // NW

    @functools.partial(
        pl.kernel,
        out_type=jax.ShapeDtypeStruct((NC, NP, D), jnp.float32),
        mesh=_mesh,
        scratch_types=(
            [
                pltpu.VMEM((2, GR, CH), jnp.int32),
                pltpu.VMEM((2, GR, CH), jnp.int32),
                pltpu.VMEM((nslot, CH, D), jnp.float32),
                pltpu.VMEM_SHARED((N, D), jnp.float32),
            ]
            + [pltpu.SemaphoreType.DMA] * (2 * nslot + 2)
        ),
        compiler_params=_sc_params,
    )
    def agg(hs_hbm, src_hbm, dst_hbm, out_hbm, si_v, di_v, rows_v,
            acc_sh, *sems):
        c = lax.axis_index("c")
        s = lax.axis_index("s")
        z16 = jnp.zeros((16,), jnp.float32)
        gsem = list(sems[:nslot])
        ssem = list(sems[nslot:2 * nslot])
        isem = list(sems[2 * nslot:])
        wid = c * NS + s

        idxd = {0: [pltpu.async_copy(src_hbm.at[wid, 0], si_v.at[0], isem[0]),
                    pltpu.async_copy(dst_hbm.at[wid, 0], di_v.at[0], isem[0])]}

        def zb(j, carry):
            for i in range(D // 16):
                rows_v[0, j, pl.ds(i * 16, 16)] = z16
            return carry

        lax.fori_loop(0, CH, zb, 0)
        zd = []
        for t in range(NRT // CH):
            zd.append(pltpu.async_copy(
                rows_v.at[0], acc_sh.at[pl.ds(s * NRT + t * CH, CH)],
                ssem[t % nslot]))
        rem = NRT % CH
        if rem:
            zd.append(pltpu.async_copy(
                rows_v.at[0, pl.ds(0, rem)],
                acc_sh.at[pl.ds(s * NRT + NRT - rem, rem)],
                ssem[(NRT // CH) % nslot]))
        for d in zd:
            d.wait()
        for d in idxd[0]:
            d.wait()
        idx_ready = {0}

        def fire(k):
            g = k // GR
            return pltpu.async_copy(
                hs_hbm.at[si_v.at[g % 2, k - g * GR]],
                rows_v.at[k % nslot], gsem[k % nslot])

        gd = {k: fire(k) for k in range(nslot - 1)}

        plsc.subcore_barrier()

        sd = {}
        for jj in range(TOT):
            g = jj // GR
            b = jj % nslot
            fk = jj + nslot - 1
            if fk < TOT:
                fg = fk // GR
                if fg not in idx_ready:
                    for d in idxd.pop(fg):
                        d.wait()
                    idx_ready.add(fg)
                if jj - 1 >= 0:
                    sd.pop(jj - 1).wait()
                gd[fk] = fire(fk)
            gd.pop(jj).wait()
            sd[jj] = pltpu.async_copy(
                rows_v.at[b], acc_sh.at[di_v.at[g % 2, jj - g * GR]],
                ssem[b], add=True)
            if jj % GR == 0 and g + 1 < NG:
                pb = (g + 1) % 2
                idxd[g + 1] = [
                    pltpu.async_copy(src_hbm.at[wid, g + 1], si_v.at[pb],
                                     isem[pb]),
                    pltpu.async_copy(dst_hbm.at[wid, g + 1], di_v.at[pb],
                                     isem[pb]),
                ]
        for jj in range(max(0, TOT - nslot), TOT):
            if jj in sd:
                sd.pop(jj).wait()

        plsc.subcore_barrier()
        pltpu.sync_copy(acc_sh.at[pl.ds(s * NRT, NRT)],
                        out_hbm.at[c, pl.ds(s * NRT, NRT)])

    return agg


_agg1 = _make_agg_flat(D1, 3)
_agg2 = _make_agg_flat(D2, 6)


def _tc1(xp, W1, degc):
    def body(x_ref, w_ref, d0_ref, d1_ref, o_ref):
        h = jnp.dot(x_ref[...], w_ref[...], preferred_element_type=jnp.float32)
        dinv = lax.rsqrt(d0_ref[...] + d1_ref[...] + 1.0)
        o_ref[...] = h * dinv

    nb = NP // 1024
    return pl.pallas_call(
        body,
        grid=(nb,),
        in_specs=[
            pl.BlockSpec((1024, D1), lambda i: (i, 0)),
            pl.BlockSpec((D1, D1), lambda i: (0, 0)),
            pl.BlockSpec((1024, 1), lambda i: (i, 0)),
            pl.BlockSpec((1024, 1), lambda i: (nb + i, 0)),
        ],
        out_specs=pl.BlockSpec((1024, D1), lambda i: (i, 0)),
        out_shape=jax.ShapeDtypeStruct((NP, D1), jnp.float32),
    )(xp, W1, degc, degc)


def _tc2(agg1, hs1, degc, W2, b1r):
    def body(a_ref, h_ref, d0_ref, d1_ref, w_ref, b_ref, o_ref):
        dinv = lax.rsqrt(d0_ref[...] + d1_ref[...] + 1.0)
        pre = (a_ref[0] + a_ref[1] + h_ref[...]) * dinv + b_ref[...]
        act = jnp.maximum(pre, 0.0)
        h2 = jnp.dot(act, w_ref[...], preferred_element_type=jnp.float32)
        o_ref[...] = h2 * dinv

    nb = NP // 1024
    return pl.pallas_call(
        body,
        grid=(nb,),
        in_specs=[
            pl.BlockSpec((NC, 1024, D1), lambda i: (0, i, 0)),
            pl.BlockSpec((1024, D1), lambda i: (i, 0)),
            pl.BlockSpec((1024, 1), lambda i: (i, 0)),
            pl.BlockSpec((1024, 1), lambda i: (nb + i, 0)),
            pl.BlockSpec((D1, D2), lambda i: (0, 0)),
            pl.BlockSpec((1, D1), lambda i: (0, 0)),
        ],
        out_specs=pl.BlockSpec((1024, D2), lambda i: (i, 0)),
        out_shape=jax.ShapeDtypeStruct((NP, D2), jnp.float32),
    )(agg1, hs1, degc, degc, W2, b1r)


def _tc3(agg2, hs2, degc, b2r):
    def body(a_ref, h_ref, d0_ref, d1_ref, b_ref, o_ref):
        dinv = lax.rsqrt(d0_ref[...] + d1_ref[...] + 1.0)
        o = (a_ref[0] + a_ref[1] + h_ref[...]) * dinv + b_ref[...]
        m = jnp.max(o, axis=1, keepdims=True)
        e = jnp.exp(o - m)
        lse = jnp.log(jnp.sum(e, axis=1, keepdims=True))
        o_ref[...] = o - m - lse

    nb = NP // 1024
    return pl.pallas_call(
        body,
        grid=(nb,),
        in_specs=[
            pl.BlockSpec((NC, 1024, D2), lambda i: (0, i, 0)),
            pl.BlockSpec((1024, D2), lambda i: (i, 0)),
            pl.BlockSpec((1024, 1), lambda i: (i, 0)),
            pl.BlockSpec((1024, 1), lambda i: (nb + i, 0)),
            pl.BlockSpec((1, D2), lambda i: (0, 0)),
        ],
        out_specs=pl.BlockSpec((1024, D2), lambda i: (i, 0)),
        out_shape=jax.ShapeDtypeStruct((NP, D2), jnp.float32),
    )(agg2, hs2, degc, degc, b2r)


def kernel(x, edge_index, eigenvectors, W1, b1, W2, b2):
    del eigenvectors
    src4 = edge_index[0].reshape(NW, NG, GR, CH)
    dst4 = edge_index[1].reshape(NW, NG, GR, CH)
    dst3 = edge_index[1].reshape(NW, ROWS_IDX // NW, CH)
    xp = jnp.pad(x, ((0, NP - N), (0, 0)))

    deg = _deg_sc(dst3)
    degc = deg.reshape(NC * NP, 1)

    hs1 = _tc1(xp, W1, degc)
    agg1 = _agg1(hs1, src4, dst4)
    hs2 = _tc2(agg1, hs1, degc, W2, b1.reshape(1, D1))
    agg2 = _agg2(hs2, src4, dst4)
    out = _tc3(agg2, hs2, degc, b2.reshape(1, D2))
    return out[:N]


# speedup vs baseline: 41.3364x; 1.0000x over previous
import functools

import jax
import jax.numpy as jnp
from jax import lax
from jax.experimental import pallas as pl
from jax.experimental.pallas import tpu as pltpu
from jax.experimental.pallas import tpu_sc as plsc

N = 10000
NP = 10240
E = 320000
D1 = 128
D2 = 64
NC = 2
NS = 16
NW = NC * NS
CH = 80
ROWS_IDX = E // CH
RPT = NP // NS
NG = 5
GR = ROWS_IDX // NW // NG
ZR = 32

_mesh = plsc.VectorSubcoreMesh(core_axis_name="c", subcore_axis_name="s")
_sc_params = pltpu.CompilerParams(use_tc_tiling_on_sc=False)


@functools.partial(
    pl.kernel,
    out_type=jax.ShapeDtypeStruct((NC, NP), jnp.float32),
    mesh=_mesh,
    scratch_types=[
        pltpu.VMEM((ROWS_IDX // NW, CH), jnp.int32),
        pltpu.VMEM((CH,), jnp.float32),
        pltpu.VMEM((RPT,), jnp.float32),
        pltpu.VMEM_SHARED((NP,), jnp.float32),
        pltpu.SemaphoreType.DMA,
        pltpu.SemaphoreType.DMA,
        pltpu.SemaphoreType.DMA,
        pltpu.SemaphoreType.DMA,
        pltpu.SemaphoreType.DMA,
    ],
    compiler_params=_sc_params,
)
def _deg_sc(dst_hbm, deg_hbm, idx_v, ones_v, zb_v, acc_sh,
            ds0, ds1, ds2, ds3, ds4):
    c = lax.axis_index("c")
    s = lax.axis_index("s")
    nrows = ROWS_IDX // NW

    o16 = jnp.ones((16,), jnp.float32)
    z16 = jnp.zeros((16,), jnp.float32)
    for i in range(CH // 16):
        ones_v[pl.ds(i * 16, 16)] = o16

    def zb(j, carry):
        zb_v[pl.ds(j * 16, 16)] = z16
        return carry

    lax.fori_loop(0, RPT // 16, zb, 0)
    pltpu.sync_copy(zb_v, acc_sh.at[pl.ds(s * RPT, RPT)])
    pltpu.sync_copy(dst_hbm.at[c * NS + s], idx_v)

    plsc.subcore_barrier()

    dsem = [ds0, ds1, ds2, ds3, ds4]
    ngrp = nrows // 25

    def step(g, carry):
        sd = {}
        for j in range(25):
            b = j % 5
            if j >= 5:
                sd.pop(j - 5).wait()
            row = g * 25 + j
            sd[j] = pltpu.async_copy(ones_v, acc_sh.at[idx_v.at[row]],
                                     dsem[b], add=True)
        for j in range(20, 25):
            sd.pop(j).wait()
        return carry

    lax.fori_loop(0, ngrp, step, 0)

    plsc.subcore_barrier()
    pltpu.sync_copy(acc_sh.at[pl.ds(s * RPT, RPT)],
                    deg_hbm.at[c, pl.ds(s * RPT, RPT)])


NRT = N // NS


def _make_agg(D, nslot):
    @functools.partial(
        pl.kernel,
        out_type=jax.ShapeDtypeStruct((NC, NP, D), jnp.float32),
        mesh=_mesh,
        scratch_types=(
            [
                pltpu.VMEM((GR, CH), jnp.int32),
                pltpu.VMEM((GR, CH), jnp.int32),
                pltpu.VMEM((nslot, CH, D), jnp.float32),
                pltpu.VMEM_SHARED((N, D), jnp.float32),
            ]
            + [pltpu.SemaphoreType.DMA] * (2 * nslot)
        ),
        compiler_params=_sc_params,
    )
    def agg(hs_hbm, src_hbm, dst_hbm, out_hbm, si_v, di_v, rows_v,
            acc_sh, *sems):
        c = lax.axis_index("c")
        s = lax.axis_index("s")
        z16 = jnp.zeros((16,), jnp.float32)
        gsem = list(sems[:nslot])
        ssem = list(sems[nslot:])

        def zb(j, carry):
            for i in range(D // 16):
                rows_v[0, j, pl.ds(i * 16, 16)] = z16
            return carry

        lax.fori_loop(0, CH, zb, 0)
        allsem = gsem + ssem
        zd = []
        for t in range(NRT // CH):
            zd.append(pltpu.async_copy(
                rows_v.at[0], acc_sh.at[pl.ds(s * NRT + t * CH, CH)],
                allsem[t % len(allsem)]))
        rem = NRT % CH
        if rem:
            zd.append(pltpu.async_copy(
                rows_v.at[0, pl.ds(0, rem)],
                acc_sh.at[pl.ds(s * NRT + NRT - rem, rem)],
                allsem[(NRT // CH) % len(allsem)]))
        for d in zd:
            d.wait()

        plsc.subcore_barrier()
        wid = c * NS + s

        def group(g, carry):
            pltpu.sync_copy(src_hbm.at[wid, g], si_v)
            pltpu.sync_copy(dst_hbm.at[wid, g], di_v)

            def fire(j):
                b = j % nslot
                return pltpu.async_copy(hs_hbm.at[si_v.at[j]], rows_v.at[b],
                                        gsem[b])

            gd = {j: fire(j) for j in range(nslot - 1)}
            sd = {}
            for j in range(GR):
                b = j % nslot
                if j + nslot - 1 < GR:
                    if j - 1 >= 0:
                        sd.pop(j - 1).wait()
                    gd[j + nslot - 1] = fire(j + nslot - 1)
                gd.pop(j).wait()
                sd[j] = pltpu.async_copy(rows_v.at[b],
                                         acc_sh.at[di_v.at[j]], ssem[b],
                                         add=True)
            for j in range(max(0, GR - nslot), GR):
                if j in sd:
                    sd.pop(j).wait()
            return carry

        lax.fori_loop(0, NG, group, 0)

        plsc.subcore_barrier()
        pltpu.sync_copy(acc_sh.at[pl.ds(s * NRT, NRT)],
                        out_hbm.at[c, pl.ds(s * NRT, NRT)])

    return agg


def _make_agg_flat(D, nslot):
    TOT = ROWS_IDX // NW

    @functools.partial(
        pl.kernel,
        out_type=jax.ShapeDtypeStruct((NC, NP, D), jnp.float32),
        mesh=_mesh,
        scratch_types=(
            [
                pltpu.VMEM((2, GR, CH), jnp.int32),
                pltpu.VMEM((2, GR, CH), jnp.int32),
                pltpu.VMEM((nslot, CH, D), jnp.float32),
                pltpu.VMEM_SHARED((N, D), jnp.float32),
            ]
            + [pltpu.SemaphoreType.DMA] * (2 * nslot + 2)
        ),
        compiler_params=_sc_params,
    )
    def agg(hs_hbm, src_hbm, dst_hbm, out_hbm, si_v, di_v, rows_v,
            acc_sh, *sems):
        c = lax.axis_index("c")
        s = lax.axis_index("s")
        z16 = jnp.zeros((16,), jnp.float32)
        gsem = list(sems[:nslot])
        ssem = list(sems[nslot:2 * nslot])
        isem = list(sems[2 * nslot:])
        wid = c * NS + s

        idxd = {0: [pltpu.async_copy(src_hbm.at[wid, 0], si_v.at[0], isem[0]),
                    pltpu.async_copy(dst_hbm.at[wid, 0], di_v.at[0], isem[0])]}

        def zb(j, carry):
            for i in range(D // 16):
                rows_v[0, j, pl.ds(i * 16, 16)] = z16
            return carry

        lax.fori_loop(0, CH, zb, 0)
        zd = []
        for t in range(NRT // CH):
            zd.append(pltpu.async_copy(
                rows_v.at[0], acc_sh.at[pl.ds(s * NRT + t * CH, CH)],
                ssem[t % nslot]))
        rem = NRT % CH
        if rem:
            zd.append(pltpu.async_copy(
                rows_v.at[0, pl.ds(0, rem)],
                acc_sh.at[pl.ds(s * NRT + NRT - rem, rem)],
                ssem[(NRT // CH) % nslot]))
        for d in zd:
            d.wait()
        for d in idxd[0]:
            d.wait()
        idx_ready = {0}

        def fire(k):
            g = k // GR
            return pltpu.async_copy(
                hs_hbm.at[si_v.at[g % 2, k - g * GR]],
                rows_v.at[k % nslot], gsem[k % nslot])

        gd = {k: fire(k) for k in range(nslot - 1)}

        plsc.subcore_barrier()

        sd = {}
        for jj in range(TOT):
            g = jj // GR
            b = jj % nslot
            fk = jj + nslot - 1
            if fk < TOT:
                fg = fk // GR
                if fg not in idx_ready:
                    for d in idxd.pop(fg):
                        d.wait()
                    idx_ready.add(fg)
                if jj - 1 >= 0:
                    sd.pop(jj - 1).wait()
                gd[fk] = fire(fk)
            gd.pop(jj).wait()
            sd[jj] = pltpu.async_copy(
                rows_v.at[b], acc_sh.at[di_v.at[g % 2, jj - g * GR]],
                ssem[b], add=True)
            if jj % GR == 0 and g + 1 < NG:
                pb = (g + 1) % 2
                idxd[g + 1] = [
                    pltpu.async_copy(src_hbm.at[wid, g + 1], si_v.at[pb],
                                     isem[pb]),
                    pltpu.async_copy(dst_hbm.at[wid, g + 1], di_v.at[pb],
                                     isem[pb]),
                ]
        for jj in range(max(0, TOT - nslot), TOT):
            if jj in sd:
                sd.pop(jj).wait()

        plsc.subcore_barrier()
        pltpu.sync_copy(acc_sh.at[pl.ds(s * NRT, NRT)],
                        out_hbm.at[c, pl.ds(s * NRT, NRT)])

    return agg


_agg1 = _make_agg_flat(D1, 3)
_agg2 = _make_agg_flat(D2, 8)


def _tc1(xp, W1, degc):
    def body(x_ref, w_ref, d0_ref, d1_ref, o_ref):
        h = jnp.dot(x_ref[...], w_ref[...], preferred_element_type=jnp.float32)
        dinv = lax.rsqrt(d0_ref[...] + d1_ref[...] + 1.0)
        o_ref[...] = h * dinv

    nb = NP // 1024
    return pl.pallas_call(
        body,
        grid=(nb,),
        in_specs=[
            pl.BlockSpec((1024, D1), lambda i: (i, 0)),
            pl.BlockSpec((D1, D1), lambda i: (0, 0)),
            pl.BlockSpec((1024, 1), lambda i: (i, 0)),
            pl.BlockSpec((1024, 1), lambda i: (nb + i, 0)),
        ],
        out_specs=pl.BlockSpec((1024, D1), lambda i: (i, 0)),
        out_shape=jax.ShapeDtypeStruct((NP, D1), jnp.float32),
    )(xp, W1, degc, degc)


def _tc2(agg1, hs1, degc, W2, b1r):
    def body(a_ref, h_ref, d0_ref, d1_ref, w_ref, b_ref, o_ref):
        dinv = lax.rsqrt(d0_ref[...] + d1_ref[...] + 1.0)
        pre = (a_ref[0] + a_ref[1] + h_ref[...]) * dinv + b_ref[...]
        act = jnp.maximum(pre, 0.0)
        h2 = jnp.dot(act, w_ref[...], preferred_element_type=jnp.float32)
        o_ref[...] = h2 * dinv

    nb = NP // 1024
    return pl.pallas_call(
        body,
        grid=(nb,),
        in_specs=[
            pl.BlockSpec((NC, 1024, D1), lambda i: (0, i, 0)),
            pl.BlockSpec((1024, D1), lambda i: (i, 0)),
            pl.BlockSpec((1024, 1), lambda i: (i, 0)),
            pl.BlockSpec((1024, 1), lambda i: (nb + i, 0)),
            pl.BlockSpec((D1, D2), lambda i: (0, 0)),
            pl.BlockSpec((1, D1), lambda i: (0, 0)),
        ],
        out_specs=pl.BlockSpec((1024, D2), lambda i: (i, 0)),
        out_shape=jax.ShapeDtypeStruct((NP, D2), jnp.float32),
    )(agg1, hs1, degc, degc, W2, b1r)


def _tc3(agg2, hs2, degc, b2r):
    def body(a_ref, h_ref, d0_ref, d1_ref, b_ref, o_ref):
        dinv = lax.rsqrt(d0_ref[...] + d1_ref[...] + 1.0)
        o = (a_ref[0] + a_ref[1] + h_ref[...]) * dinv + b_ref[...]
        m = jnp.max(o, axis=1, keepdims=True)
        e = jnp.exp(o - m)
        lse = jnp.log(jnp.sum(e, axis=1, keepdims=True))
        o_ref[...] = o - m - lse

    nb = NP // 1024
    return pl.pallas_call(
        body,
        grid=(nb,),
        in_specs=[
            pl.BlockSpec((NC, 1024, D2), lambda i: (0, i, 0)),
            pl.BlockSpec((1024, D2), lambda i: (i, 0)),
            pl.BlockSpec((1024, 1), lambda i: (i, 0)),
            pl.BlockSpec((1024, 1), lambda i: (nb + i, 0)),
            pl.BlockSpec((1, D2), lambda i: (0, 0)),
        ],
        out_specs=pl.BlockSpec((1024, D2), lambda i: (i, 0)),
        out_shape=jax.ShapeDtypeStruct((NP, D2), jnp.float32),
    )(agg2, hs2, degc, degc, b2r)


def kernel(x, edge_index, eigenvectors, W1, b1, W2, b2):
    del eigenvectors
    src4 = edge_index[0].reshape(NW, NG, GR, CH)
    dst4 = edge_index[1].reshape(NW, NG, GR, CH)
    dst3 = edge_index[1].reshape(NW, ROWS_IDX // NW, CH)
    xp = jnp.pad(x, ((0, NP - N), (0, 0)))

    deg = _deg_sc(dst3)
    degc = deg.reshape(NC * NP, 1)

    hs1 = _tc1(xp, W1, degc)
    agg1 = _agg1(hs1, src4, dst4)
    hs2 = _tc2(agg1, hs1, degc, W2, b1.reshape(1, D1))
    agg2 = _agg2(hs2, src4, dst4)
    out = _tc3(agg2, hs2, degc, b2.reshape(1, D2))
    return out[:N]


# speedup vs baseline: 41.4065x; 1.0017x over previous
import functools

import jax
import jax.numpy as jnp
from jax import lax
from jax.experimental import pallas as pl
from jax.experimental.pallas import tpu as pltpu
from jax.experimental.pallas import tpu_sc as plsc

N = 10000
NP = 10240
E = 320000
D1 = 128
D2 = 64
NC = 2
NS = 16
NW = NC * NS
CH = 80
ROWS_IDX = E // CH
RPT = NP // NS
NG = 5
GR = ROWS_IDX // NW // NG
ZR = 32

_mesh = plsc.VectorSubcoreMesh(core_axis_name="c", subcore_axis_name="s")
_sc_params = pltpu.CompilerParams(use_tc_tiling_on_sc=False)


@functools.partial(
    pl.kernel,
    out_type=jax.ShapeDtypeStruct((NC, NP), jnp.float32),
    mesh=_mesh,
    scratch_types=[
        pltpu.VMEM((ROWS_IDX // NW, CH), jnp.int32),
        pltpu.VMEM((CH,), jnp.float32),
        pltpu.VMEM((RPT,), jnp.float32),
        pltpu.VMEM_SHARED((NP,), jnp.float32),
        pltpu.SemaphoreType.DMA,
        pltpu.SemaphoreType.DMA,
        pltpu.SemaphoreType.DMA,
        pltpu.SemaphoreType.DMA,
        pltpu.SemaphoreType.DMA,
    ],
    compiler_params=_sc_params,
)
def _deg_sc(dst_hbm, deg_hbm, idx_v, ones_v, zb_v, acc_sh,
            ds0, ds1, ds2, ds3, ds4):
    c = lax.axis_index("c")
    s = lax.axis_index("s")
    nrows = ROWS_IDX // NW

    o16 = jnp.ones((16,), jnp.float32)
    z16 = jnp.zeros((16,), jnp.float32)
    for i in range(CH // 16):
        ones_v[pl.ds(i * 16, 16)] = o16

    def zb(j, carry):
        zb_v[pl.ds(j * 16, 16)] = z16
        return carry

    lax.fori_loop(0, RPT // 16, zb, 0)
    pltpu.sync_copy(zb_v, acc_sh.at[pl.ds(s * RPT, RPT)])
    pltpu.sync_copy(dst_hbm.at[c * NS + s], idx_v)

    plsc.subcore_barrier()

    dsem = [ds0, ds1, ds2, ds3, ds4]
    ngrp = nrows // 25

    def step(g, carry):
        sd = {}
        for j in range(25):
            b = j % 5
            if j >= 5:
                sd.pop(j - 5).wait()
            row = g * 25 + j
            sd[j] = pltpu.async_copy(ones_v, acc_sh.at[idx_v.at[row]],
                                     dsem[b], add=True)
        for j in range(20, 25):
            sd.pop(j).wait()
        return carry

    lax.fori_loop(0, ngrp, step, 0)

    plsc.subcore_barrier()
    pltpu.sync_copy(acc_sh.at[pl.ds(s * RPT, RPT)],
                    deg_hbm.at[c, pl.ds(s * RPT, RPT)])


NRT = N // NS


def _make_agg_flat(D, nslot):
    TOT = ROWS_IDX // NW

    @functools.partial(
        pl.kernel,
        out_type=jax.ShapeDtypeStruct((NC, N, D), jnp.float32),
        mesh=_mesh,
        scratch_types=(
            [
                pltpu.VMEM((2, GR, CH), jnp.int32),
                pltpu.VMEM((2, GR, CH), jnp.int32),
                pltpu.VMEM((nslot, CH, D), jnp.float32),
                pltpu.VMEM_SHARED((N, D), jnp.float32),
            ]
            + [pltpu.SemaphoreType.DMA] * (2 * nslot + 2)
        ),
        compiler_params=_sc_params,
    )
    def agg(hs_hbm, src_hbm, dst_hbm, out_hbm, si_v, di_v, rows_v,
            acc_sh, *sems):
        c = lax.axis_index("c")
        s = lax.axis_index("s")
        z16 = jnp.zeros((16,), jnp.float32)
        gsem = list(sems[:nslot])
        ssem = list(sems[nslot:2 * nslot])
        isem = list(sems[2 * nslot:])
        wid = c * NS + s

        idxd = {0: [pltpu.async_copy(src_hbm.at[wid, 0], si_v.at[0], isem[0]),
                    pltpu.async_copy(dst_hbm.at[wid, 0], di_v.at[0], isem[0])]}

        def zb(j, carry):
            for i in range(D // 16):
                rows_v[0, j, pl.ds(i * 16, 16)] = z16
            return carry

        lax.fori_loop(0, CH, zb, 0)
        zd = []
        for t in range(NRT // CH):
            zd.append(pltpu.async_copy(
                rows_v.at[0], acc_sh.at[pl.ds(s * NRT + t * CH, CH)],
                ssem[t % nslot]))
        rem = NRT % CH
        if rem:
            zd.append(pltpu.async_copy(
                rows_v.at[0, pl.ds(0, rem)],
                acc_sh.at[pl.ds(s * NRT + NRT - rem, rem)],
                ssem[(NRT // CH) % nslot]))
        for d in zd:
            d.wait()
        for d in idxd[0]:
            d.wait()
        idx_ready = {0}

        def fire(k):
            g = k // GR
            return pltpu.async_copy(
                hs_hbm.at[si_v.at[g % 2, k - g * GR]],
                rows_v.at[k % nslot], gsem[k % nslot])

        gd = {k: fire(k) for k in range(nslot - 1)}

        plsc.subcore_barrier()

        sd = {}
        for jj in range(TOT):
            g = jj // GR
            b = jj % nslot
            fk = jj + nslot - 1
            if fk < TOT:
                fg = fk // GR
                if fg not in idx_ready:
                    for d in idxd.pop(fg):
                        d.wait()
                    idx_ready.add(fg)
                if jj - 1 >= 0:
                    sd.pop(jj - 1).wait()
                gd[fk] = fire(fk)
            gd.pop(jj).wait()
            sd[jj] = pltpu.async_copy(
                rows_v.at[b], acc_sh.at[di_v.at[g % 2, jj - g * GR]],
                ssem[b], add=True)
            if jj % GR == 0 and g + 1 < NG:
                pb = (g + 1) % 2
                idxd[g + 1] = [
                    pltpu.async_copy(src_hbm.at[wid, g + 1], si_v.at[pb],
                                     isem[pb]),
                    pltpu.async_copy(dst_hbm.at[wid, g + 1], di_v.at[pb],
                                     isem[pb]),
                ]
        for jj in range(max(0, TOT - nslot), TOT):
            if jj in sd:
                sd.pop(jj).wait()

        plsc.subcore_barrier()
        pltpu.sync_copy(acc_sh.at[pl.ds(s * NRT, NRT)],
                        out_hbm.at[c, pl.ds(s * NRT, NRT)])

    return agg


_agg1 = _make_agg_flat(D1, 3)
_agg2 = _make_agg_flat(D2, 6)


RB = 1000


def _tc1(x, W1, degc):
    def body(x_ref, w_ref, d0_ref, d1_ref, o_ref):
        h = jnp.dot(x_ref[...], w_ref[...], preferred_element_type=jnp.float32)
        dinv = lax.rsqrt(d0_ref[...] + d1_ref[...] + 1.0)
        o_ref[...] = h * dinv

    nb = N // RB
    return pl.pallas_call(
        body,
        grid=(nb,),
        in_specs=[
            pl.BlockSpec((RB, D1), lambda i: (i, 0)),
            pl.BlockSpec((D1, D1), lambda i: (0, 0)),
            pl.BlockSpec((RB, 1), lambda i: (i, 0)),
            pl.BlockSpec((RB, 1), lambda i: (nb + i, 0)),
        ],
        out_specs=pl.BlockSpec((RB, D1), lambda i: (i, 0)),
        out_shape=jax.ShapeDtypeStruct((N, D1), jnp.float32),
    )(x, W1, degc, degc)


def _tc2(agg1, hs1, degc, W2, b1r):
    def body(a_ref, h_ref, d0_ref, d1_ref, w_ref, b_ref, o_ref):
        dinv = lax.rsqrt(d0_ref[...] + d1_ref[...] + 1.0)
        pre = (a_ref[0] + a_ref[1] + h_ref[...]) * dinv + b_ref[...]
        act = jnp.maximum(pre, 0.0)
        h2 = jnp.dot(act, w_ref[...], preferred_element_type=jnp.float32)
        o_ref[...] = h2 * dinv

    nb = N // RB
    return pl.pallas_call(
        body,
        grid=(nb,),
        in_specs=[
            pl.BlockSpec((NC, RB, D1), lambda i: (0, i, 0)),
            pl.BlockSpec((RB, D1), lambda i: (i, 0)),
            pl.BlockSpec((RB, 1), lambda i: (i, 0)),
            pl.BlockSpec((RB, 1), lambda i: (nb + i, 0)),
            pl.BlockSpec((D1, D2), lambda i: (0, 0)),
            pl.BlockSpec((1, D1), lambda i: (0, 0)),
        ],
        out_specs=pl.BlockSpec((RB, D2), lambda i: (i, 0)),
        out_shape=jax.ShapeDtypeStruct((N, D2), jnp.float32),
    )(agg1, hs1, degc, degc, W2, b1r)


def _tc3(agg2, hs2, degc, b2r):
    def body(a_ref, h_ref, d0_ref, d1_ref, b_ref, o_ref):
        dinv = lax.rsqrt(d0_ref[...] + d1_ref[...] + 1.0)
        o = (a_ref[0] + a_ref[1] + h_ref[...]) * dinv + b_ref[...]
        m = jnp.max(o, axis=1, keepdims=True)
        e = jnp.exp(o - m)
        lse = jnp.log(jnp.sum(e, axis=1, keepdims=True))
        o_ref[...] = o - m - lse

    nb = N // RB
    return pl.pallas_call(
        body,
        grid=(nb,),
        in_specs=[
            pl.BlockSpec((NC, RB, D2), lambda i: (0, i, 0)),
            pl.BlockSpec((RB, D2), lambda i: (i, 0)),
            pl.BlockSpec((RB, 1), lambda i: (i, 0)),
            pl.BlockSpec((RB, 1), lambda i: (nb + i, 0)),
            pl.BlockSpec((1, D2), lambda i: (0, 0)),
        ],
        out_specs=pl.BlockSpec((RB, D2), lambda i: (i, 0)),
        out_shape=jax.ShapeDtypeStruct((N, D2), jnp.float32),
    )(agg2, hs2, degc, degc, b2r)


def kernel(x, edge_index, eigenvectors, W1, b1, W2, b2):
    del eigenvectors
    src4 = edge_index[0].reshape(NW, NG, GR, CH)
    dst4 = edge_index[1].reshape(NW, NG, GR, CH)
    dst3 = edge_index[1].reshape(NW, ROWS_IDX // NW, CH)

    deg = _deg_sc(dst3)
    degc = deg[:, :N].reshape(NC * N, 1)

    hs1 = _tc1(x, W1, degc)
    agg1 = _agg1(hs1, src4, dst4)
    hs2 = _tc2(agg1, hs1, degc, W2, b1.reshape(1, D1))
    agg2 = _agg2(hs2, src4, dst4)
    return _tc3(agg2, hs2, degc, b2.reshape(1, D2))
